# Initial kernel scaffold; baseline (speedup 1.0000x reference)
import jax
import jax.numpy as jnp
from jax.experimental import pallas as pl


def kernel(relation, all_y, relation_emb_weight):
    raise NotImplementedError("write your pallas kernel here")


# speedup vs baseline: 5.8169x; 5.8169x over previous
import functools

import jax
import jax.numpy as jnp
from jax import lax
from jax.experimental import pallas as pl
from jax.experimental.pallas import tpu as pltpu
from jax.experimental.pallas import tpu_sc as plsc

_D = 64
_B_L = 16384
_B_Y = 16384 * 50
_NW = 32
_CHUNK = 128
_GRP = 4
_ROWS_G = _GRP * _CHUNK
_L_CH = _B_L // (_NW * _CHUNK)
_Y_CH = _B_Y // (_NW * _CHUNK)


def _body(relidx, allyidx, table, out_l, out_y, idx_l, idx_y, rows, sem):
    wid = lax.axis_index("s") * 2 + lax.axis_index("c")

    pltpu.sync_copy(relidx.at[pl.ds(wid * _L_CH, _L_CH)], idx_l)
    pltpu.sync_copy(allyidx.at[pl.ds(wid * _Y_CH, _Y_CH)], idx_y)

    cs = [
        pltpu.async_copy(table.at[idx_l.at[k]],
                         rows.at[pl.ds(k * _CHUNK, _CHUNK)], sem)
        for k in range(_GRP)
    ]
    for c in cs:
        c.wait()
    pltpu.sync_copy(rows, out_l.at[pl.ds(wid * _ROWS_G, _ROWS_G)])

    @pl.loop(0, _Y_CH // _GRP)
    def _(g):
        gs = [
            pltpu.async_copy(table.at[idx_y.at[g * _GRP + k]],
                             rows.at[pl.ds(k * _CHUNK, _CHUNK)], sem)
            for k in range(_GRP)
        ]
        for c in gs:
            c.wait()
        pltpu.sync_copy(
            rows, out_y.at[pl.ds((wid * _Y_CH + g * _GRP) * _CHUNK, _ROWS_G)])


@functools.partial(jax.jit, donate_argnums=())
def kernel(relation, all_y, relation_emb_weight):
    relidx = relation.reshape(_B_L // _CHUNK, _CHUNK)
    allyidx = all_y.reshape(_B_Y // _CHUNK, _CHUNK)
    mesh = plsc.VectorSubcoreMesh(core_axis_name="c", subcore_axis_name="s")
    out_l, out_y = pl.kernel(
        _body,
        out_type=(
            jax.ShapeDtypeStruct((_B_L, _D), jnp.float32),
            jax.ShapeDtypeStruct((_B_Y, _D), jnp.float32),
        ),
        mesh=mesh,
        compiler_params=pltpu.CompilerParams(use_tc_tiling_on_sc=False),
        scratch_types=[
            pltpu.VMEM((_L_CH, _CHUNK), jnp.int32),
            pltpu.VMEM((_Y_CH, _CHUNK), jnp.int32),
            pltpu.VMEM((_ROWS_G, _D), jnp.float32),
            pltpu.SemaphoreType.DMA,
        ],
    )(relidx, allyidx, relation_emb_weight)
    return (out_l.reshape(_B_L, 1, _D), relation_emb_weight,
            out_y.reshape(_B_L, 50, _D))


# speedup vs baseline: 5.9997x; 1.0314x over previous
import functools

import jax
import jax.numpy as jnp
from jax import lax
from jax.experimental import pallas as pl
from jax.experimental.pallas import tpu as pltpu
from jax.experimental.pallas import tpu_sc as plsc

_D = 64
_B_L = 16384
_B_Y = 16384 * 50
_NW = 32
_CHUNK = 128
_GRP = 2
_NBUF = 4
_ROWS_G = _GRP * _CHUNK
_L_CH = _B_L // (_NW * _CHUNK)
_Y_CH = _B_Y // (_NW * _CHUNK)
_Y_GRPS = _Y_CH // _GRP
_Y_ITERS = _Y_GRPS // _NBUF


def _body(relidx, allyidx, table, out_l, out_y,
          idx_l, idx_y, bufs, isem, gsems, osems):
    wid = lax.axis_index("s") * 2 + lax.axis_index("c")

    pltpu.sync_copy(relidx.at[pl.ds(wid * _L_CH, _L_CH)], idx_l)
    idx_cp = pltpu.async_copy(
        allyidx.at[pl.ds(wid * _Y_CH, _Y_CH)], idx_y, isem)

    lcs = [
        pltpu.async_copy(table.at[idx_l.at[g * _GRP + k]],
                         bufs[g].at[pl.ds(k * _CHUNK, _CHUNK)], gsems[g])
        for g in range(2) for k in range(_GRP)
    ]
    for c in lcs:
        c.wait()
    ocs = [
        pltpu.async_copy(
            bufs[g], out_l.at[pl.ds((wid * _L_CH + g * _GRP) * _CHUNK,
                                    _ROWS_G)], osems[g])
        for g in range(2)
    ]
    idx_cp.wait()
    for c in ocs:
        c.wait()

    @pl.loop(0, _Y_ITERS)
    def _(t):
        g0 = t * _NBUF
        gcs = [
            pltpu.async_copy(table.at[idx_y.at[(g0 + b) * _GRP + k]],
                             bufs[b].at[pl.ds(k * _CHUNK, _CHUNK)], gsems[b])
            for b in range(_NBUF) for k in range(_GRP)
        ]
        wcs = []
        for b in range(_NBUF):
            gcs[2 * b].wait()
            gcs[2 * b + 1].wait()
            wcs.append(pltpu.async_copy(
                bufs[b],
                out_y.at[pl.ds((wid * _Y_CH + (g0 + b) * _GRP) * _CHUNK,
                               _ROWS_G)], osems[b]))
        for c in wcs:
            c.wait()


@functools.partial(jax.jit, donate_argnums=())
def kernel(relation, all_y, relation_emb_weight):
    relidx = relation.reshape(_B_L // _CHUNK, _CHUNK)
    allyidx = all_y.reshape(_B_Y // _CHUNK, _CHUNK)
    mesh = plsc.VectorSubcoreMesh(core_axis_name="c", subcore_axis_name="s")
    out_l, out_y = pl.kernel(
        _body,
        out_type=(
            jax.ShapeDtypeStruct((_B_L, _D), jnp.float32),
            jax.ShapeDtypeStruct((_B_Y, _D), jnp.float32),
        ),
        mesh=mesh,
        compiler_params=pltpu.CompilerParams(use_tc_tiling_on_sc=False),
        scratch_types=[
            pltpu.VMEM((_L_CH, _CHUNK), jnp.int32),
            pltpu.VMEM((_Y_CH, _CHUNK), jnp.int32),
            [pltpu.VMEM((_ROWS_G, _D), jnp.float32) for _ in range(_NBUF)],
            pltpu.SemaphoreType.DMA,
            [pltpu.SemaphoreType.DMA for _ in range(_NBUF)],
            [pltpu.SemaphoreType.DMA for _ in range(_NBUF)],
        ],
    )(relidx, allyidx, relation_emb_weight)
    return (out_l.reshape(_B_L, 1, _D), relation_emb_weight,
            out_y.reshape(_B_L, 50, _D))
